# baseline (speedup 1.0000x reference)
import functools

import jax
import jax.numpy as jnp
from jax import lax
from jax.experimental import pallas as pl
from jax.experimental.pallas import tpu as pltpu
from jax.experimental.pallas import tpu_sc as plsc

EMBED_DIM = 128
VOCAB = 11
BATCH = 16384
_VPAD = 16
_NPAIR = 128
_PAIR_DIM = 2 * EMBED_DIM
_LROW = 128

_NC = 2
_NS = 16
_NW = _NC * _NS
_NPAIRS = BATCH // 2
_P_PER_W = _NPAIRS // _NW


def _prep_body(tpad_ref, gamma_ref, beta_ref, lab_ref, pairs_ref, pidx_ref):
    x = tpad_ref[...]
    mean = jnp.mean(x, axis=1, keepdims=True)
    d = x - mean
    var = jnp.mean(d * d, axis=1, keepdims=True)
    norm = d * lax.rsqrt(var + 1e-5) * gamma_ref[...] + beta_ref[...]

    k = lax.broadcasted_iota(jnp.int32, (_NPAIR, 1), 0)
    cols = lax.broadcasted_iota(jnp.int32, (_NPAIR, _VPAD), 1)
    oh_i = jnp.where(cols == k // VOCAB, 1.0, 0.0)
    oh_j = jnp.where(cols == k % VOCAB, 1.0, 0.0)
    pairs_ref[:, :EMBED_DIM] = jnp.dot(
        oh_i, norm, preferred_element_type=jnp.float32,
        precision=lax.Precision.HIGHEST)
    pairs_ref[:, EMBED_DIM:] = jnp.dot(
        oh_j, norm, preferred_element_type=jnp.float32,
        precision=lax.Precision.HIGHEST)

    r = lax.broadcasted_iota(jnp.int32, (_LROW, _LROW // 2), 0)
    t = lax.broadcasted_iota(jnp.int32, (_LROW, _LROW // 2), 1)
    w = jnp.where(r == 2 * t, float(VOCAB), 0.0) + jnp.where(
        r == 2 * t + 1, 1.0, 0.0)
    labf = lab_ref[...].astype(jnp.float32)
    pidx_ref[...] = jnp.dot(
        labf, w, preferred_element_type=jnp.float32,
        precision=lax.Precision.HIGHEST).astype(jnp.int32)


_tc_prep = pl.pallas_call(
    _prep_body,
    out_shape=(
        jax.ShapeDtypeStruct((_NPAIR, _PAIR_DIM), jnp.float32),
        jax.ShapeDtypeStruct((_LROW, _LROW // 2), jnp.int32),
    ),
)


_sc_mesh = plsc.VectorSubcoreMesh(core_axis_name="c", subcore_axis_name="s")


@functools.partial(
    pl.kernel,
    mesh=_sc_mesh,
    out_type=jax.ShapeDtypeStruct((_NPAIRS, _PAIR_DIM), jnp.float32),
    scratch_types=[
        pltpu.VMEM((_P_PER_W // 2,), jnp.int32),
        pltpu.VMEM((_P_PER_W // 2,), jnp.int32),
        pltpu.VMEM((_P_PER_W // 2, _PAIR_DIM), jnp.float32),
        pltpu.VMEM((_P_PER_W // 2, _PAIR_DIM), jnp.float32),
        pltpu.SemaphoreType.DMA,
        pltpu.SemaphoreType.DMA,
    ],
)
def _sc_gather(pairs_hbm, pidx_hbm, out_hbm, pidx_a, pidx_b, rows_a, rows_b,
               sem_a, sem_b):
    wid = lax.axis_index("s") * _NC + lax.axis_index("c")
    base = wid * _P_PER_W
    half = _P_PER_W // 2
    pltpu.sync_copy(pidx_hbm.at[pl.ds(base, half)], pidx_a)
    pltpu.sync_copy(pidx_hbm.at[pl.ds(base + half, half)], pidx_b)
    ca = pltpu.async_copy(pairs_hbm.at[pidx_a], rows_a, sem_a)
    cb = pltpu.async_copy(pairs_hbm.at[pidx_b], rows_b, sem_b)
    ca.wait()
    pltpu.sync_copy(rows_a, out_hbm.at[pl.ds(base, half)])
    cb.wait()
    pltpu.sync_copy(rows_b, out_hbm.at[pl.ds(base + half, half)])


def kernel(labels, table, gamma, beta):
    tpad = jnp.zeros((_VPAD, EMBED_DIM), jnp.float32).at[:VOCAB].set(table)
    lab2d = labels.astype(jnp.int32).reshape(_LROW, _LROW)
    pairs, pidx = _tc_prep(
        tpad, gamma.reshape(1, EMBED_DIM), beta.reshape(1, EMBED_DIM), lab2d
    )
    out = _sc_gather(pairs, pidx.reshape(_NPAIRS))
    return out.reshape(BATCH, 1, EMBED_DIM)


# speedup vs baseline: 1.6105x; 1.6105x over previous
import functools

import jax
import jax.numpy as jnp
from jax import lax
from jax.experimental import pallas as pl
from jax.experimental.pallas import tpu as pltpu
from jax.experimental.pallas import tpu_sc as plsc

EMBED_DIM = 128
VOCAB = 11
BATCH = 16384
_VPAD = 16

_NC = 2
_NS = 16
_NW = _NC * _NS
_B_PER_W = BATCH // _NW
_LANES = 16


def _ln_body(tpad_ref, gamma_ref, beta_ref, out_ref):
    x = tpad_ref[...]
    mean = jnp.mean(x, axis=1, keepdims=True)
    d = x - mean
    var = jnp.mean(d * d, axis=1, keepdims=True)
    out_ref[...] = d * lax.rsqrt(var + 1e-5) * gamma_ref[...] + beta_ref[...]


_normalize_table = pl.pallas_call(
    _ln_body,
    out_shape=jax.ShapeDtypeStruct((_VPAD, EMBED_DIM), jnp.float32),
)


_sc_mesh = plsc.VectorSubcoreMesh(core_axis_name="c", subcore_axis_name="s")


@functools.partial(
    pl.kernel,
    mesh=_sc_mesh,
    out_type=jax.ShapeDtypeStruct((BATCH, EMBED_DIM), jnp.float32),
    scratch_types=[
        pltpu.VMEM((_VPAD, EMBED_DIM), jnp.float32),
        pltpu.VMEM((_B_PER_W,), jnp.int32),
        pltpu.VMEM((_B_PER_W, EMBED_DIM), jnp.float32),
    ],
)
def _sc_build(norm_hbm, lab_hbm, out_hbm, table_v, lab_v, rows_v):
    wid = lax.axis_index("s") * _NC + lax.axis_index("c")
    base = wid * _B_PER_W
    pltpu.sync_copy(norm_hbm, table_v)
    pltpu.sync_copy(lab_hbm.at[pl.ds(base, _B_PER_W)], lab_v)

    def body(c, _):
        lv = lab_v[pl.ds(c * _LANES, _LANES)]
        for l in range(_LANES):
            p = lv[l]
            rows_v[c * _LANES + l, :] = table_v[p, :]
        return _

    lax.fori_loop(0, _B_PER_W // _LANES, body, None)
    pltpu.sync_copy(rows_v, out_hbm.at[pl.ds(base, _B_PER_W)])


def kernel(labels, table, gamma, beta):
    tpad = jnp.zeros((_VPAD, EMBED_DIM), jnp.float32).at[:VOCAB].set(table)
    norm = _normalize_table(
        tpad, gamma.reshape(1, EMBED_DIM), beta.reshape(1, EMBED_DIM)
    )
    out = _sc_build(norm, labels.astype(jnp.int32))
    return out.reshape(BATCH, 1, EMBED_DIM)
